# Initial kernel scaffold; baseline (speedup 1.0000x reference)
import jax
import jax.numpy as jnp
from jax.experimental import pallas as pl


def kernel(indices, table):
    raise NotImplementedError("write your pallas kernel here")


# speedup vs baseline: 1.4591x; 1.4591x over previous
import functools

import jax
import jax.numpy as jnp
from jax import lax
from jax.experimental import pallas as pl
from jax.experimental.pallas import tpu as pltpu
from jax.experimental.pallas import tpu_sc as plsc

_DIM = 32
_NW = 32
_SUB = 128
_N_SUB = 8
_CHUNK = _SUB * _N_SUB


def _make_gather(n_rows):
    b_per_w = n_rows // _NW
    n_chunks = b_per_w // _CHUNK
    mesh = plsc.VectorSubcoreMesh(core_axis_name="c", subcore_axis_name="s")

    @functools.partial(
        pl.kernel,
        out_type=jax.ShapeDtypeStruct((n_rows, _DIM), jnp.float32),
        mesh=mesh,
        scratch_types=[
            pltpu.VMEM((_N_SUB, _SUB), jnp.int32),
            pltpu.VMEM((_CHUNK, _DIM), jnp.float32),
            pltpu.SemaphoreType.DMA,
        ],
        compiler_params=pltpu.CompilerParams(use_tc_tiling_on_sc=False),
    )
    def gather_kernel(idx_hbm, table_hbm, out_hbm, idx_v, rows_v, sem):
        wid = lax.axis_index("s") * 2 + lax.axis_index("c")
        idx_row0 = wid * (b_per_w // _SUB)
        out_row0 = wid * b_per_w

        def body(ci, carry):
            pltpu.sync_copy(
                idx_hbm.at[pl.ds(idx_row0 + ci * _N_SUB, _N_SUB)], idx_v)
            copies = [
                pltpu.async_copy(
                    table_hbm.at[idx_v.at[j]],
                    rows_v.at[pl.ds(j * _SUB, _SUB)],
                    sem,
                )
                for j in range(_N_SUB)
            ]
            for c in copies:
                c.wait()
            pltpu.sync_copy(
                rows_v, out_hbm.at[pl.ds(out_row0 + ci * _CHUNK, _CHUNK)])
            return carry

        lax.fori_loop(0, n_chunks, body, 0)

    return gather_kernel


def kernel(indices, table):
    b, h = indices.shape
    n = b * h
    idx2d = indices.reshape(n // _SUB, _SUB)
    out = _make_gather(n)(idx2d, table)
    return out.reshape(b, h, _DIM)


# speedup vs baseline: 1.4947x; 1.0244x over previous
import functools

import jax
import jax.numpy as jnp
from jax import lax
from jax.experimental import pallas as pl
from jax.experimental.pallas import tpu as pltpu
from jax.experimental.pallas import tpu_sc as plsc

_DIM = 32
_NW = 32
_SUB = 128
_N_SUB = 10
_CHUNK = _SUB * _N_SUB
_NBUF = 2


def _make_gather(n_rows):
    b_per_w = n_rows // _NW
    n_chunks = b_per_w // _CHUNK
    assert n_chunks % _NBUF == 0
    mesh = plsc.VectorSubcoreMesh(core_axis_name="c", subcore_axis_name="s")

    @functools.partial(
        pl.kernel,
        out_type=jax.ShapeDtypeStruct((n_rows, _DIM), jnp.float32),
        mesh=mesh,
        scratch_types=[
            pltpu.VMEM((_NBUF, _N_SUB, _SUB), jnp.int32),
            pltpu.VMEM((_NBUF, _CHUNK, _DIM), jnp.float32),
            pltpu.SemaphoreType.DMA((_NBUF,)),
            pltpu.SemaphoreType.DMA((_NBUF,)),
            pltpu.SemaphoreType.DMA((_NBUF,)),
        ],
        compiler_params=pltpu.CompilerParams(use_tc_tiling_on_sc=False),
    )
    def gather_kernel(idx_hbm, table_hbm, out_hbm, idx_v, rows_v,
                      idx_sem, gat_sem, wb_sem):
        wid = lax.axis_index("s") * 2 + lax.axis_index("c")
        idx_row0 = wid * (b_per_w // _SUB)
        out_row0 = wid * b_per_w

        def idx_copy(g, b):
            return pltpu.make_async_copy(
                idx_hbm.at[pl.ds(idx_row0 + g * _N_SUB, _N_SUB)],
                idx_v.at[b], idx_sem.at[b])

        def wb_copy(g, b):
            return pltpu.make_async_copy(
                rows_v.at[b],
                out_hbm.at[pl.ds(out_row0 + g * _CHUNK, _CHUNK)],
                wb_sem.at[b])

        idx_copy(0, 0).start()

        def body(gg, carry):
            for b in range(_NBUF):
                g = gg * _NBUF + b
                idx_copy(g, b).wait()

                @pl.when(g + 1 < n_chunks)
                def _():
                    idx_copy(g + 1, (b + 1) % _NBUF).start()

                @pl.when(g >= _NBUF)
                def _():
                    wb_copy(g - _NBUF, b).wait()

                copies = [
                    pltpu.async_copy(
                        table_hbm.at[idx_v.at[b, j]],
                        rows_v.at[b].at[pl.ds(j * _SUB, _SUB)],
                        gat_sem.at[b],
                    )
                    for j in range(_N_SUB)
                ]
                for c in copies:
                    c.wait()
                wb_copy(g, b).start()
            return carry

        lax.fori_loop(0, n_chunks // _NBUF, body, 0)
        for b in range(_NBUF):
            wb_copy(n_chunks - _NBUF + b, b).wait()

    return gather_kernel


def kernel(indices, table):
    b, h = indices.shape
    n = b * h
    idx2d = indices.reshape(n // _SUB, _SUB)
    out = _make_gather(n)(idx2d, table)
    return out.reshape(b, h, _DIM)
